# baseline (speedup 1.0000x reference)
import functools

import jax
import jax.numpy as jnp
from jax import lax
from jax.experimental import pallas as pl
from jax.experimental.pallas import tpu as pltpu
from jax.experimental.pallas import tpu_sc as plsc

NC = 2
NS = 16
NW = NC * NS
CHUNK = 128
D = 128
NB = 5
LOOKAHEAD = 3


@functools.cache
def _emb_kernel(n_idx: int):
  b_per_w = n_idx // NW
  n_chunks = b_per_w // CHUNK
  mesh = plsc.VectorSubcoreMesh(
      core_axis_name="c", subcore_axis_name="s", num_cores=NC, num_subcores=NS
  )

  @functools.partial(
      pl.kernel,
      out_type=jax.ShapeDtypeStruct((n_idx, D), jnp.float32),
      mesh=mesh,
      scratch_types=[
          pltpu.VMEM((b_per_w,), jnp.int32),
          pltpu.VMEM((NB * CHUNK, D), jnp.float32),
          pltpu.SemaphoreType.DMA((NB,)),
          pltpu.SemaphoreType.DMA((NB,)),
      ],
  )
  def k(words_hbm, table_hbm, out_hbm, idx_v, rows_v, gsem, osem):
    wid = lax.axis_index("s") * NC + lax.axis_index("c")
    base = wid * b_per_w
    pltpu.sync_copy(words_hbm.at[pl.ds(base, b_per_w)], idx_v)

    def slot_ref(s):
      return rows_v.at[pl.ds(s * CHUNK, CHUNK)]

    def fire_gather(c, s):
      pltpu.async_copy(
          table_hbm.at[idx_v.at[pl.ds(c * CHUNK, CHUNK)]],
          slot_ref(s),
          gsem.at[s],
      )

    def wait_write(s):
      pltpu.make_async_copy(
          table_hbm.at[pl.ds(0, CHUNK)], slot_ref(s), osem.at[s]
      ).wait()

    for c in range(LOOKAHEAD):
      fire_gather(c, c % NB)

    def body(j, _):
      s = lax.rem(j, NB)
      pltpu.make_async_copy(
          table_hbm.at[pl.ds(0, CHUNK)], slot_ref(s), gsem.at[s]
      ).wait()
      pltpu.async_copy(
          slot_ref(s), out_hbm.at[pl.ds(base + j * CHUNK, CHUNK)], osem.at[s]
      )
      nxt = j + LOOKAHEAD
      ns = lax.rem(nxt, NB)

      @pl.when(nxt < n_chunks)
      def _():
        @pl.when(nxt >= NB)
        def _():
          wait_write(ns)

        fire_gather(nxt, ns)

      return 0

    lax.fori_loop(0, n_chunks, body, 0)

    for b in range(NB):
      wait_write(b)

  return k


def kernel(words, table):
  b, h = words.shape
  idx = words.T.reshape(-1).astype(jnp.int32)
  out = _emb_kernel(idx.size)(idx, table.astype(jnp.float32))
  return out.reshape(h, b, D).transpose(1, 0, 2)
